# Initial kernel scaffold; baseline (speedup 1.0000x reference)
import jax
import jax.numpy as jnp
from jax.experimental import pallas as pl


def kernel(logits):
    raise NotImplementedError("write your pallas kernel here")


# speedup vs baseline: 53.1107x; 53.1107x over previous
import jax
import jax.numpy as jnp
from jax import lax
from jax.experimental import pallas as pl
from jax.experimental.pallas import tpu as pltpu
from jax.experimental.pallas import tpu_sc as plsc

R = 128
N = 32768
L = 16
NC = 2
NS = 16
NW = NC * NS
RPW = R // NW
NCH = N // L
NB = 2048
NBCH = NB // L
MARGIN = 2.0
BIG = 3.0e38

_mesh = plsc.VectorSubcoreMesh(
    core_axis_name="c", subcore_axis_name="s",
    num_cores=NC, num_subcores=NS)


def _body(x_hbm, out_hbm, row_v, cnt_v, sum_v):
    wid = lax.axis_index("s") * NC + lax.axis_index("c")
    iota_f = lax.convert_element_type(lax.iota(jnp.int32, L), jnp.float32)
    ones = jnp.ones((L,), jnp.float32)
    zeros = jnp.zeros((L,), jnp.float32)

    def bcast(s):
        return jnp.broadcast_to(s, (L,))

    def locate(start, width):
        w = width * (1.0 / NB)
        invw = NB / width
        end = start + width

        def zero_body(i, carry):
            cnt_v[pl.ds(i * L, L)] = zeros
            sum_v[pl.ds(i * L, L)] = zeros
            return carry

        lax.fori_loop(0, NBCH, zero_body, 0)

        def hist_body(i, carry):
            khi, shi = carry
            v = row_v[pl.ds(i * L, L)]
            rel = v - start
            m_in = (v > start) & (v < end)
            m_hi = v >= end
            idx = jnp.clip(rel * invw, 0.0, NB - 1.0).astype(jnp.int32)
            plsc.addupdate_scatter(cnt_v, [idx], ones, mask=m_in)
            plsc.addupdate_scatter(sum_v, [idx], rel, mask=m_in)
            khi = khi + jnp.where(m_hi, 1.0, 0.0)
            shi = shi + jnp.where(m_hi, rel, 0.0)
            return khi, shi

        khi_v, shi_v = lax.fori_loop(0, NCH, hist_body, (zeros, zeros))
        k_hi = bcast(jnp.sum(khi_v))
        s_hi = bcast(jnp.sum(shi_v))

        def scan_body(t, carry):
            ck, cs, ntrue = carry
            base = (NBCH - 1 - t) * L
            c = cnt_v[pl.ds(base, L)]
            s = sum_v[pl.ds(base, L)]
            sufc = lax.rev(jnp.cumsum(lax.rev(c, (0,))), (0,)) + ck
            sufs = lax.rev(jnp.cumsum(lax.rev(s, (0,))), (0,)) + cs
            jf = (lax.convert_element_type(base, jnp.float32) + iota_f) * w
            f = (s_hi + sufs) - (k_hi + sufc) * jf
            ntrue = ntrue + jnp.where(f > 1.0, 1.0, 0.0)
            ck = ck + bcast(jnp.sum(c))
            cs = cs + bcast(jnp.sum(s))
            return ck, cs, ntrue

        _, _, ntrue_v = lax.fori_loop(0, NBCH, scan_body,
                                      (zeros, zeros, zeros))
        j1 = jnp.maximum(bcast(jnp.sum(ntrue_v)) - 1.0, 0.0)
        jlo = jnp.clip(j1 - MARGIN, 0.0, NB - 1.0)
        jhi = jnp.clip(j1 + MARGIN, 0.0, NB - 1.0)
        return start + jlo * w, (jhi + 1.0 - jlo) * w

    def final_tau(o, o_end):
        def body(i, carry):
            khi, shi, cin, sin, mn, mx = carry
            v = row_v[pl.ds(i * L, L)]
            rel = v - o
            m_hi = v >= o_end
            m_in = (v > o) & (v < o_end)
            khi = khi + jnp.where(m_hi, 1.0, 0.0)
            shi = shi + jnp.where(m_hi, rel, 0.0)
            cin = cin + jnp.where(m_in, 1.0, 0.0)
            sin = sin + jnp.where(m_in, rel, 0.0)
            mn = jnp.minimum(mn, jnp.where(m_in, rel, BIG))
            mx = jnp.maximum(mx, jnp.where(m_in, rel, -BIG))
            return khi, shi, cin, sin, mn, mx

        init = (zeros, zeros, zeros, zeros,
                jnp.full((L,), BIG, jnp.float32),
                jnp.full((L,), -BIG, jnp.float32))
        khi, shi, cin, sin, mn, mx = lax.fori_loop(0, NCH, body, init)
        k_hi = bcast(jnp.sum(khi))
        s_hi = bcast(jnp.sum(shi))
        c_in = bcast(jnp.sum(cin))
        s_in = bcast(jnp.sum(sin))
        minr = bcast(jnp.min(mn))
        maxr = bcast(jnp.max(mx))
        d_a = (s_hi - 1.0) / jnp.maximum(k_hi, 1.0)
        d_b = (s_hi + s_in - 1.0) / jnp.maximum(k_hi + c_in, 1.0)
        d_c = (s_hi + maxr - 1.0) / (k_hi + 1.0)
        valid_a = (k_hi > 0.0) & ((c_in == 0.0) | (d_a >= maxr))
        valid_b = ((k_hi + c_in) > 0.0) & ((c_in == 0.0) | (d_b < minr))
        valid_c = (c_in > 0.0) & (d_c < maxr) & ((c_in == 1.0) | (d_c >= minr))
        d = jnp.where(valid_a, d_a,
                      jnp.where(valid_b, d_b,
                                jnp.where(valid_c, d_c, d_b)))
        return o + d

    def row_body(rr, carry):
        r = wid * RPW + rr
        pltpu.sync_copy(x_hbm.at[r], row_v)

        def max_body(i, acc):
            return jnp.maximum(acc, row_v[pl.ds(i * L, L)])

        m = bcast(jnp.max(lax.fori_loop(0, NCH, max_body,
                                        jnp.full((L,), -BIG, jnp.float32))))
        s1, w1 = locate(m - 1.001, jnp.full((L,), 1.002, jnp.float32))
        s2, w2 = locate(s1, w1)
        tau = final_tau(s2, s2 + w2)

        def out_body(i, c):
            v = row_v[pl.ds(i * L, L)]
            row_v[pl.ds(i * L, L)] = jnp.maximum(v - tau, 0.0)
            return c

        lax.fori_loop(0, NCH, out_body, 0)
        pltpu.sync_copy(row_v, out_hbm.at[r])
        return carry

    lax.fori_loop(0, RPW, row_body, 0)


_sparsemax_sc = pl.kernel(
    _body,
    out_type=jax.ShapeDtypeStruct((R, N), jnp.float32),
    mesh=_mesh,
    compiler_params=pltpu.CompilerParams(needs_layout_passes=False),
    scratch_types=[
        pltpu.VMEM((N,), jnp.float32),
        pltpu.VMEM((NB,), jnp.float32),
        pltpu.VMEM((NB,), jnp.float32),
    ],
)


@jax.jit
def kernel(logits):
    return _sparsemax_sc(logits)


# speedup vs baseline: 60.6465x; 1.1419x over previous
import jax
import jax.numpy as jnp
from jax import lax
from jax.experimental import pallas as pl
from jax.experimental.pallas import tpu as pltpu
from jax.experimental.pallas import tpu_sc as plsc

R = 128
N = 32768
L = 16
NC = 2
NS = 16
NW = NC * NS
RPW = R // NW
NCH = N // L
NB = 2048
NBCH = NB // L
MARGIN = 2.0
BIG = 3.0e38

_mesh = plsc.VectorSubcoreMesh(
    core_axis_name="c", subcore_axis_name="s",
    num_cores=NC, num_subcores=NS)


def _body(x_hbm, out_hbm, row_v, cnt_v, sum_v):
    wid = lax.axis_index("s") * NC + lax.axis_index("c")
    iota_f = lax.convert_element_type(lax.iota(jnp.int32, L), jnp.float32)
    ones = jnp.ones((L,), jnp.float32)
    zeros = jnp.zeros((L,), jnp.float32)

    def bcast(s):
        return jnp.broadcast_to(s, (L,))

    def locate(start, width):
        w = width * (1.0 / NB)
        invw = NB / width
        end = start + width

        def zero_body(i, carry):
            cnt_v[pl.ds(i * L, L)] = zeros
            sum_v[pl.ds(i * L, L)] = zeros
            return carry

        lax.fori_loop(0, NBCH, zero_body, 0, unroll=4)

        def hist_body(i, carry):
            khi, shi = carry
            v = row_v[pl.ds(i * L, L)]
            rel = v - start
            m_in = (v > start) & (v < end)
            m_hi = v >= end
            idx = jnp.clip(rel * invw, 0.0, NB - 1.0).astype(jnp.int32)
            plsc.addupdate_scatter(cnt_v, [idx], ones, mask=m_in)
            plsc.addupdate_scatter(sum_v, [idx], rel, mask=m_in)
            khi = khi + jnp.where(m_hi, 1.0, 0.0)
            shi = shi + jnp.where(m_hi, rel, 0.0)
            return khi, shi

        khi_v, shi_v = lax.fori_loop(0, NCH, hist_body, (zeros, zeros),
                                     unroll=8)
        k_hi = bcast(jnp.sum(khi_v))
        s_hi = bcast(jnp.sum(shi_v))

        def scan_body(t, carry):
            ck, cs, ntrue = carry
            base = (NBCH - 1 - t) * L
            c = cnt_v[pl.ds(base, L)]
            s = sum_v[pl.ds(base, L)]
            sufc = lax.rev(jnp.cumsum(lax.rev(c, (0,))), (0,)) + ck
            sufs = lax.rev(jnp.cumsum(lax.rev(s, (0,))), (0,)) + cs
            jf = (lax.convert_element_type(base, jnp.float32) + iota_f) * w
            f = (s_hi + sufs) - (k_hi + sufc) * jf
            ntrue = ntrue + jnp.where(f > 1.0, 1.0, 0.0)
            ck = ck + bcast(jnp.sum(c))
            cs = cs + bcast(jnp.sum(s))
            return ck, cs, ntrue

        _, _, ntrue_v = lax.fori_loop(0, NBCH, scan_body,
                                      (zeros, zeros, zeros), unroll=4)
        j1 = jnp.maximum(bcast(jnp.sum(ntrue_v)) - 1.0, 0.0)
        jlo = jnp.clip(j1 - MARGIN, 0.0, NB - 1.0)
        jhi = jnp.clip(j1 + MARGIN, 0.0, NB - 1.0)
        return start + jlo * w, (jhi + 1.0 - jlo) * w

    def final_tau(o, o_end):
        def body(i, carry):
            khi, shi, cin, sin, mn, mx = carry
            v = row_v[pl.ds(i * L, L)]
            rel = v - o
            m_hi = v >= o_end
            m_in = (v > o) & (v < o_end)
            khi = khi + jnp.where(m_hi, 1.0, 0.0)
            shi = shi + jnp.where(m_hi, rel, 0.0)
            cin = cin + jnp.where(m_in, 1.0, 0.0)
            sin = sin + jnp.where(m_in, rel, 0.0)
            mn = jnp.minimum(mn, jnp.where(m_in, rel, BIG))
            mx = jnp.maximum(mx, jnp.where(m_in, rel, -BIG))
            return khi, shi, cin, sin, mn, mx

        init = (zeros, zeros, zeros, zeros,
                jnp.full((L,), BIG, jnp.float32),
                jnp.full((L,), -BIG, jnp.float32))
        khi, shi, cin, sin, mn, mx = lax.fori_loop(0, NCH, body, init,
                                                   unroll=8)
        k_hi = bcast(jnp.sum(khi))
        s_hi = bcast(jnp.sum(shi))
        c_in = bcast(jnp.sum(cin))
        s_in = bcast(jnp.sum(sin))
        minr = bcast(jnp.min(mn))
        maxr = bcast(jnp.max(mx))
        d_a = (s_hi - 1.0) / jnp.maximum(k_hi, 1.0)
        d_b = (s_hi + s_in - 1.0) / jnp.maximum(k_hi + c_in, 1.0)
        d_c = (s_hi + maxr - 1.0) / (k_hi + 1.0)
        valid_a = (k_hi > 0.0) & ((c_in == 0.0) | (d_a >= maxr))
        valid_b = ((k_hi + c_in) > 0.0) & ((c_in == 0.0) | (d_b < minr))
        valid_c = (c_in > 0.0) & (d_c < maxr) & ((c_in == 1.0) | (d_c >= minr))
        d = jnp.where(valid_a, d_a,
                      jnp.where(valid_b, d_b,
                                jnp.where(valid_c, d_c, d_b)))
        return o + d

    def row_body(rr, carry):
        r = wid * RPW + rr
        pltpu.sync_copy(x_hbm.at[r], row_v)

        def max_body(i, acc):
            return jnp.maximum(acc, row_v[pl.ds(i * L, L)])

        m = bcast(jnp.max(lax.fori_loop(0, NCH, max_body,
                                        jnp.full((L,), -BIG, jnp.float32),
                                        unroll=8)))
        s1, w1 = locate(m - 1.001, jnp.full((L,), 1.002, jnp.float32))
        s2, w2 = locate(s1, w1)
        tau = final_tau(s2, s2 + w2)

        def out_body(i, c):
            v = row_v[pl.ds(i * L, L)]
            row_v[pl.ds(i * L, L)] = jnp.maximum(v - tau, 0.0)
            return c

        lax.fori_loop(0, NCH, out_body, 0, unroll=8)
        pltpu.sync_copy(row_v, out_hbm.at[r])
        return carry

    lax.fori_loop(0, RPW, row_body, 0)


_sparsemax_sc = pl.kernel(
    _body,
    out_type=jax.ShapeDtypeStruct((R, N), jnp.float32),
    mesh=_mesh,
    compiler_params=pltpu.CompilerParams(needs_layout_passes=False),
    scratch_types=[
        pltpu.VMEM((N,), jnp.float32),
        pltpu.VMEM((NB,), jnp.float32),
        pltpu.VMEM((NB,), jnp.float32),
    ],
)


@jax.jit
def kernel(logits):
    return _sparsemax_sc(logits)


# speedup vs baseline: 152.2354x; 2.5102x over previous
import jax
import jax.numpy as jnp
from jax import lax
from jax.experimental import pallas as pl
from jax.experimental.pallas import tpu as pltpu
from jax.experimental.pallas import tpu_sc as plsc

R = 128
N = 32768
L = 16
NC = 2
NS = 16
NW = NC * NS
RPW = R // NW
NCH = N // L
NBIS = 25
PAD = 1e-6
BIG = 3.0e38

_mesh = plsc.VectorSubcoreMesh(
    core_axis_name="c", subcore_axis_name="s",
    num_cores=NC, num_subcores=NS)


def _body(x_hbm, out_hbm, row_v, cand_v):
    wid = lax.axis_index("s") * NC + lax.axis_index("c")
    ones_i = jnp.ones((L,), jnp.int32)
    zeros = jnp.zeros((L,), jnp.float32)

    def bcast(s):
        return jnp.broadcast_to(s, (L,))

    def scal(v):
        return lax.squeeze(lax.slice(v, (0,), (1,)), (0,))

    def final_tau(o, o_end, nch_c):
        def body(i, carry):
            khi, shi, cin, sin, mn, mx = carry
            v = cand_v[pl.ds(i * L, L)]
            rel = v - o
            m_hi = v >= o_end
            m_in = (v > o) & (v < o_end)
            khi = khi + jnp.where(m_hi, 1.0, 0.0)
            shi = shi + jnp.where(m_hi, rel, 0.0)
            cin = cin + jnp.where(m_in, 1.0, 0.0)
            sin = sin + jnp.where(m_in, rel, 0.0)
            mn = jnp.minimum(mn, jnp.where(m_in, rel, BIG))
            mx = jnp.maximum(mx, jnp.where(m_in, rel, -BIG))
            return khi, shi, cin, sin, mn, mx

        init = (zeros, zeros, zeros, zeros,
                jnp.full((L,), BIG, jnp.float32),
                jnp.full((L,), -BIG, jnp.float32))
        khi, shi, cin, sin, mn, mx = lax.fori_loop(0, nch_c, body, init)
        k_hi = bcast(jnp.sum(khi))
        s_hi = bcast(jnp.sum(shi))
        c_in = bcast(jnp.sum(cin))
        s_in = bcast(jnp.sum(sin))
        minr = bcast(jnp.min(mn))
        maxr = bcast(jnp.max(mx))
        d_a = (s_hi - 1.0) / jnp.maximum(k_hi, 1.0)
        d_b = (s_hi + s_in - 1.0) / jnp.maximum(k_hi + c_in, 1.0)
        d_c = (s_hi + maxr - 1.0) / (k_hi + 1.0)
        valid_a = (k_hi > 0.0) & ((c_in == 0.0) | (d_a >= maxr))
        valid_b = ((k_hi + c_in) > 0.0) & ((c_in == 0.0) | (d_b < minr))
        valid_c = (c_in > 0.0) & (d_c < maxr) & ((c_in == 1.0) | (d_c >= minr))
        d = jnp.where(valid_a, d_a,
                      jnp.where(valid_b, d_b,
                                jnp.where(valid_c, d_c, d_b)))
        return o + d

    def row_body(rr, carry):
        r = wid * RPW + rr
        pltpu.sync_copy(x_hbm.at[r], row_v)

        def max_body(i, acc):
            return jnp.maximum(acc, row_v[pl.ds(i * L, L)])

        m = bcast(jnp.max(lax.fori_loop(0, NCH, max_body,
                                        jnp.full((L,), -BIG, jnp.float32),
                                        unroll=8)))
        start = m - 1.001

        def compact_body(i, off):
            v = row_v[pl.ds(i * L, L)]
            msk = v > start
            plsc.store_compressed(cand_v.at[pl.ds(off, L)], v, mask=msk)
            return off + scal(plsc.all_reduce_population_count(msk))

        off = lax.fori_loop(0, NCH, compact_body, jnp.zeros((), jnp.int32),
                            unroll=4)
        cand_v[pl.ds(off, L)] = jnp.full((L,), -BIG, jnp.float32)
        nch_c = lax.shift_right_logical(off + (L - 1), 4)

        def bis_body(_, carry):
            lo, hi = carry
            mid = 0.5 * (lo + hi)

            def acc_body(i, acc):
                v = cand_v[pl.ds(i * L, L)]
                return acc + jnp.where(v > mid, v - mid, 0.0)

            f = bcast(jnp.sum(lax.fori_loop(0, nch_c, acc_body, zeros)))
            gt = f > 1.0
            return jnp.where(gt, mid, lo), jnp.where(gt, hi, mid)

        lo, hi = lax.fori_loop(0, NBIS, bis_body,
                               (start, m + 0.001))
        tau = final_tau(lo - PAD, hi + PAD, nch_c)

        def out_body(i, c):
            v = row_v[pl.ds(i * L, L)]
            row_v[pl.ds(i * L, L)] = jnp.maximum(v - tau, 0.0)
            return c

        lax.fori_loop(0, NCH, out_body, 0, unroll=8)
        pltpu.sync_copy(row_v, out_hbm.at[r])
        return carry

    lax.fori_loop(0, RPW, row_body, 0)


_sparsemax_sc = pl.kernel(
    _body,
    out_type=jax.ShapeDtypeStruct((R, N), jnp.float32),
    mesh=_mesh,
    compiler_params=pltpu.CompilerParams(needs_layout_passes=False),
    scratch_types=[
        pltpu.VMEM((N,), jnp.float32),
        pltpu.VMEM((N + L,), jnp.float32),
    ],
)


@jax.jit
def kernel(logits):
    return _sparsemax_sc(logits)
